# baseline (speedup 1.0000x reference)
import jax
import jax.numpy as jnp
from jax import lax
from jax.experimental import pallas as pl
from jax.experimental.pallas import tpu as pltpu
from jax.experimental.pallas import tpu_sc as plsc

_N = 50000
_D = 256
_S = 512
_NC = 2
_NS = 16
_NW = _NC * _NS
_R = 112
_L = 16
_SEG_PER_TILE = _S // _NW
_SP = 528

_BS = 2000
_G = _N // _BS

_GATHER_DNUMS = lax.GatherDimensionNumbers(
    offset_dims=(), collapsed_slice_dims=(0,), start_index_map=(0,))


def _lane_all_sum(v):
    iota = lax.iota(jnp.int32, _L)
    for sh in (8, 4, 2, 1):
        idx = (iota ^ sh).reshape(_L, 1)
        v = v + lax.gather(v, idx, _GATHER_DNUMS, (1,),
                           mode=lax.GatherScatterMode.PROMISE_IN_BOUNDS)
    return v


def _starts_fn(seg_ref, st_ref):
    i = pl.program_id(0)

    @pl.when(i == 0)
    def _init():
        st_ref[...] = jnp.zeros_like(st_ref)

    seg = seg_ref[0]
    lanes = lax.broadcasted_iota(jnp.int32, (_BS, _SP), 1)
    ind = (seg < lanes).astype(jnp.int32)
    st_ref[...] += jnp.sum(ind, axis=0, keepdims=True)


def _sc_body(x_hbm, seg_hbm, st_hbm, w_hbm, b_hbm, out_hbm,
             xbuf, bbuf, stv, wvec, bvec, acc):
    cid = lax.axis_index("c")
    sid = lax.axis_index("s")
    wid = sid * _NC + cid

    pltpu.sync_copy(w_hbm, wvec)
    pltpu.sync_copy(b_hbm, bvec)
    pltpu.sync_copy(st_hbm, stv)

    zero = jnp.zeros((_L,), jnp.float32)

    def _zrow(i, _):
        for j in range(_D // _L):
            acc[i, pl.ds(j * _L, _L)] = zero
        return 0

    lax.fori_loop(0, _SEG_PER_TILE, _zrow, 0)

    base = wid * _SEG_PER_TILE
    a = stv[pl.ds(base, _L)][0]
    e = stv[pl.ds(base + _SEG_PER_TILE, _L)][0]
    a8 = (a >> 3) << 3
    nch = (e - a8 + _R - 1) // _R

    def _chunk(c, _):
        cs = pl.multiple_of(jnp.minimum(a8 + c * _R, _N - _R), 8)
        pltpu.sync_copy(x_hbm.at[pl.ds(cs, _R)], xbuf)
        pltpu.sync_copy(seg_hbm.at[pl.ds(cs, _R)], bbuf)
        lo = jnp.maximum(a8 + c * _R, a)

        def _grp(gi, _):
            segv = bbuf[pl.ds(gi * _L, _L)]
            for k in range(_L):
                r = gi * _L + k
                g = cs + r
                valid = jnp.logical_and(g >= lo, g < e)
                s_off = jnp.clip(segv[k] - base, 0, _SEG_PER_TILE - 1)
                acc16 = xbuf[r, pl.ds(0, _L)] * wvec[pl.ds(0, _L)]
                for j in range(1, _D // _L):
                    acc16 = acc16 + xbuf[r, pl.ds(j * _L, _L)] * wvec[pl.ds(j * _L, _L)]
                zv = _lane_all_sum(acc16) + bvec[...]
                wgt = (1.0 / (1.0 + jnp.exp(-zv))) * valid.astype(jnp.float32)
                for j in range(_D // _L):
                    sl = pl.ds(j * _L, _L)
                    plsc.addupdate(acc.at[s_off, sl], xbuf[r, sl] * wgt)
            return 0

        lax.fori_loop(0, _R // _L, _grp, 0)
        return 0

    lax.fori_loop(0, nch, _chunk, 0)
    pltpu.sync_copy(acc, out_hbm.at[pl.ds(base, _SEG_PER_TILE)])


def kernel(x, batch, W, b):
    seg = batch.astype(jnp.int32)
    wr = W.reshape(_D).astype(jnp.float32)
    br = jnp.broadcast_to(b.astype(jnp.float32), (_L,))

    starts = pl.pallas_call(
        _starts_fn,
        grid=(_G,),
        in_specs=[pl.BlockSpec((1, _BS, 1), lambda i: (i, 0, 0))],
        out_specs=pl.BlockSpec((1, _SP), lambda i: (0, 0)),
        out_shape=jax.ShapeDtypeStruct((1, _SP), jnp.int32),
        compiler_params=pltpu.CompilerParams(
            dimension_semantics=("arbitrary",),
        ),
    )(seg.reshape(_G, _BS, 1))

    mesh = plsc.VectorSubcoreMesh(
        core_axis_name="c", subcore_axis_name="s",
        num_cores=_NC, num_subcores=_NS)
    sc_fn = pl.kernel(
        _sc_body,
        out_type=jax.ShapeDtypeStruct((_S, _D), jnp.float32),
        mesh=mesh,
        scratch_types=[
            pltpu.VMEM((_R, _D), jnp.float32),
            pltpu.VMEM((_R,), jnp.int32),
            pltpu.VMEM((_SP,), jnp.int32),
            pltpu.VMEM((_D,), jnp.float32),
            pltpu.VMEM((_L,), jnp.float32),
            pltpu.VMEM((_SEG_PER_TILE, _D), jnp.float32),
        ],
    )
    return sc_fn(x, seg, starts.reshape(_SP), wr, br)


# speedup vs baseline: 2.1984x; 2.1984x over previous
import jax
import jax.numpy as jnp
from jax import lax
from jax.experimental import pallas as pl
from jax.experimental.pallas import tpu as pltpu
from jax.experimental.pallas import tpu_sc as plsc

_N = 50000
_D = 256
_S = 512
_NC = 2
_NS = 16
_NW = _NC * _NS
_R = 112
_L = 16
_SEG_PER_TILE = _S // _NW
_SP = 528

_BS = 2000
_G = _N // _BS

_GATHER_DNUMS = lax.GatherDimensionNumbers(
    offset_dims=(), collapsed_slice_dims=(0,), start_index_map=(0,))


def _lane_all_sum(v):
    iota = lax.iota(jnp.int32, _L)
    for sh in (8, 4, 2, 1):
        idx = (iota ^ sh).reshape(_L, 1)
        v = v + lax.gather(v, idx, _GATHER_DNUMS, (1,),
                           mode=lax.GatherScatterMode.PROMISE_IN_BOUNDS)
    return v


def _starts_fn(seg_ref, st_ref):
    i = pl.program_id(0)

    @pl.when(i == 0)
    def _init():
        st_ref[...] = jnp.zeros_like(st_ref)

    seg = seg_ref[0]
    lanes = lax.broadcasted_iota(jnp.int32, (_BS, _SP), 1)
    ind = (seg < lanes).astype(jnp.int32)
    st_ref[...] += jnp.sum(ind, axis=0, keepdims=True)


def _sc_body(x_hbm, seg_hbm, st_hbm, w_hbm, b_hbm, out_hbm,
             xbuf, bbuf, stv, wvec, bvec, acc):
    cid = lax.axis_index("c")
    sid = lax.axis_index("s")
    wid = sid * _NC + cid

    pltpu.sync_copy(w_hbm, wvec)
    pltpu.sync_copy(b_hbm, bvec)
    pltpu.sync_copy(st_hbm, stv)

    zero = jnp.zeros((_L,), jnp.float32)

    def _zrow(i, _):
        for j in range(_D // _L):
            acc[i, pl.ds(j * _L, _L)] = zero
        return 0

    lax.fori_loop(0, _SEG_PER_TILE, _zrow, 0)

    base = wid * _SEG_PER_TILE
    a = stv[pl.ds(base, _L)][0]
    e = stv[pl.ds(base + _SEG_PER_TILE, _L)][0]
    a8 = (a >> 3) << 3
    nch = (e - a8 + _R - 1) // _R

    wregs = tuple(wvec[pl.ds(j * _L, _L)] for j in range(_D // _L))
    bv = bvec[...]

    def _chunk(c, wr_):
        cs = pl.multiple_of(jnp.minimum(a8 + c * _R, _N - _R), 8)
        pltpu.sync_copy(x_hbm.at[pl.ds(cs, _R)], xbuf)
        pltpu.sync_copy(seg_hbm.at[pl.ds(cs, _R)], bbuf)
        lo = jnp.maximum(a8 + c * _R, a)

        def _grp(gi, wr):
            segv = bbuf[pl.ds(gi * _L, _L)]
            for k in range(_L):
                r = gi * _L + k
                g = cs + r
                valid = jnp.logical_and(g >= lo, g < e)
                s_off = jnp.clip(segv[k] - base, 0, _SEG_PER_TILE - 1)
                xr = [xbuf[r, pl.ds(j * _L, _L)] for j in range(_D // _L)]
                p = [xr[j] * wr[j] for j in range(_D // _L)]
                while len(p) > 1:
                    p = [p[2 * i] + p[2 * i + 1] for i in range(len(p) // 2)]
                zv = _lane_all_sum(p[0]) + bv
                wgt = (1.0 / (1.0 + jnp.exp(-zv))) * valid.astype(jnp.float32)
                for j in range(_D // _L):
                    plsc.addupdate(acc.at[s_off, pl.ds(j * _L, _L)], xr[j] * wgt)
            return wr

        return lax.fori_loop(0, _R // _L, _grp, wr_)

    lax.fori_loop(0, nch, _chunk, wregs)
    pltpu.sync_copy(acc, out_hbm.at[pl.ds(base, _SEG_PER_TILE)])


def kernel(x, batch, W, b):
    seg = batch.astype(jnp.int32)
    wr = W.reshape(_D).astype(jnp.float32)
    br = jnp.broadcast_to(b.astype(jnp.float32), (_L,))

    starts = pl.pallas_call(
        _starts_fn,
        grid=(_G,),
        in_specs=[pl.BlockSpec((1, _BS, 1), lambda i: (i, 0, 0))],
        out_specs=pl.BlockSpec((1, _SP), lambda i: (0, 0)),
        out_shape=jax.ShapeDtypeStruct((1, _SP), jnp.int32),
        compiler_params=pltpu.CompilerParams(
            dimension_semantics=("arbitrary",),
        ),
    )(seg.reshape(_G, _BS, 1))

    mesh = plsc.VectorSubcoreMesh(
        core_axis_name="c", subcore_axis_name="s",
        num_cores=_NC, num_subcores=_NS)
    sc_fn = pl.kernel(
        _sc_body,
        out_type=jax.ShapeDtypeStruct((_S, _D), jnp.float32),
        mesh=mesh,
        scratch_types=[
            pltpu.VMEM((_R, _D), jnp.float32),
            pltpu.VMEM((_R,), jnp.int32),
            pltpu.VMEM((_SP,), jnp.int32),
            pltpu.VMEM((_D,), jnp.float32),
            pltpu.VMEM((_L,), jnp.float32),
            pltpu.VMEM((_SEG_PER_TILE, _D), jnp.float32),
        ],
    )
    return sc_fn(x, seg, starts.reshape(_SP), wr, br)


# speedup vs baseline: 2.2072x; 1.0040x over previous
import jax
import jax.numpy as jnp
from jax import lax
from jax.experimental import pallas as pl
from jax.experimental.pallas import tpu as pltpu
from jax.experimental.pallas import tpu_sc as plsc

_N = 50000
_D = 256
_S = 512
_NC = 2
_NS = 16
_NW = _NC * _NS
_R = 112
_L = 16
_SEG_PER_TILE = _S // _NW
_SP = 528

_BS = 2000
_G = _N // _BS

_GATHER_DNUMS = lax.GatherDimensionNumbers(
    offset_dims=(), collapsed_slice_dims=(0,), start_index_map=(0,))


def _lane_all_sum(v):
    iota = lax.iota(jnp.int32, _L)
    for sh in (8, 4, 2, 1):
        idx = (iota ^ sh).reshape(_L, 1)
        v = v + lax.gather(v, idx, _GATHER_DNUMS, (1,),
                           mode=lax.GatherScatterMode.PROMISE_IN_BOUNDS)
    return v


def _starts_fn(seg_ref, st_ref):
    i = pl.program_id(0)

    @pl.when(i == 0)
    def _init():
        st_ref[...] = jnp.zeros_like(st_ref)

    seg = seg_ref[0]
    lanes = lax.broadcasted_iota(jnp.int32, (_BS, _SP), 1)
    ind = (seg < lanes).astype(jnp.int32)
    st_ref[...] += jnp.sum(ind, axis=0, keepdims=True)


def _sc_body(x_hbm, seg_hbm, st_hbm, w_hbm, b_hbm, out_hbm,
             xbuf, bbuf, stv, wvec, bvec, acc):
    cid = lax.axis_index("c")
    sid = lax.axis_index("s")
    wid = sid * _NC + cid

    pltpu.sync_copy(w_hbm, wvec)
    pltpu.sync_copy(b_hbm, bvec)
    pltpu.sync_copy(st_hbm, stv)

    zero = jnp.zeros((_L,), jnp.float32)

    def _zrow(i, _):
        for j in range(_D // _L):
            acc[i, pl.ds(j * _L, _L)] = zero
        return 0

    lax.fori_loop(0, _SEG_PER_TILE, _zrow, 0)

    base = wid * _SEG_PER_TILE
    a = stv[pl.ds(base, _L)][0]
    e = stv[pl.ds(base + _SEG_PER_TILE, _L)][0]
    a8 = (a >> 3) << 3
    nch = (e - a8 + _R - 1) // _R

    wregs = tuple(wvec[pl.ds(j * _L, _L)] for j in range(_D // _L))
    bv = bvec[...]

    def _chunk(c, wr_):
        cs = pl.multiple_of(jnp.minimum(a8 + c * _R, _N - _R), 8)
        pltpu.sync_copy(x_hbm.at[pl.ds(cs, _R)], xbuf)
        pltpu.sync_copy(seg_hbm.at[pl.ds(cs, _R)], bbuf.at[pl.ds(0, _R)])
        lo = jnp.maximum(a8 + c * _R, a)

        def _grp(gi, wr):
            segv = bbuf[pl.ds(gi * _L, _L)]
            giv = lax.iota(jnp.int32, _L) + (cs + gi * _L)
            mask = jnp.logical_and(giv >= lo, giv < e)
            mf = jnp.where(mask, 1.0, 0.0)
            for k in range(_L):
                r = gi * _L + k
                s_off = jnp.clip(segv[k] - base, 0, _SEG_PER_TILE - 1)
                xr = [xbuf[r, pl.ds(j * _L, _L)] for j in range(_D // _L)]
                p = [xr[j] * wr[j] for j in range(_D // _L)]
                while len(p) > 1:
                    p = [p[2 * i] + p[2 * i + 1] for i in range(len(p) // 2)]
                zv = _lane_all_sum(p[0]) + bv
                wgt = (mf[k] / (1.0 + jnp.exp(-zv)))
                for j in range(_D // _L):
                    plsc.addupdate(acc.at[s_off, pl.ds(j * _L, _L)], xr[j] * wgt)
            return wr

        return plsc.parallel_loop(0, _R // _L, carry=wr_)(_grp)

    lax.fori_loop(0, nch, _chunk, wregs)
    pltpu.sync_copy(acc, out_hbm.at[pl.ds(base, _SEG_PER_TILE)])


def kernel(x, batch, W, b):
    seg = batch.astype(jnp.int32)
    wr = W.reshape(_D).astype(jnp.float32)
    br = jnp.broadcast_to(b.astype(jnp.float32), (_L,))

    starts = pl.pallas_call(
        _starts_fn,
        grid=(_G,),
        in_specs=[pl.BlockSpec((1, _BS, 1), lambda i: (i, 0, 0))],
        out_specs=pl.BlockSpec((1, _SP), lambda i: (0, 0)),
        out_shape=jax.ShapeDtypeStruct((1, _SP), jnp.int32),
        compiler_params=pltpu.CompilerParams(
            dimension_semantics=("arbitrary",),
        ),
    )(seg.reshape(_G, _BS, 1))

    mesh = plsc.VectorSubcoreMesh(
        core_axis_name="c", subcore_axis_name="s",
        num_cores=_NC, num_subcores=_NS)
    sc_fn = pl.kernel(
        _sc_body,
        out_type=jax.ShapeDtypeStruct((_S, _D), jnp.float32),
        mesh=mesh,
        scratch_types=[
            pltpu.VMEM((_R, _D), jnp.float32),
            pltpu.VMEM((_R + _L,), jnp.int32),
            pltpu.VMEM((_SP,), jnp.int32),
            pltpu.VMEM((_D,), jnp.float32),
            pltpu.VMEM((_L,), jnp.float32),
            pltpu.VMEM((_SEG_PER_TILE, _D), jnp.float32),
        ],
    )
    return sc_fn(x, seg, starts.reshape(_SP), wr, br)


# speedup vs baseline: 2.3790x; 1.0778x over previous
import jax
import jax.numpy as jnp
from jax import lax
from jax.experimental import pallas as pl
from jax.experimental.pallas import tpu as pltpu
from jax.experimental.pallas import tpu_sc as plsc

_N = 50000
_D = 256
_S = 512
_NC = 2
_NS = 16
_NW = _NC * _NS
_R = 112
_L = 16
_SEG_PER_TILE = _S // _NW
_SP = 528
_BS = 2000
_G = _N // _BS


def _tc_fn(xb_ref, seg_ref, w_ref, b_ref, st_ref, wgt_ref):
    i = pl.program_id(0)

    @pl.when(i == 0)
    def _init():
        st_ref[...] = jnp.zeros_like(st_ref)

    seg = seg_ref[0]
    lanes = lax.broadcasted_iota(jnp.int32, (_BS, _SP), 1)
    st_ref[...] += jnp.sum((seg < lanes).astype(jnp.int32), axis=0,
                           keepdims=True)
    xb = xb_ref[...]
    z = jnp.sum(xb * w_ref[...], axis=1, keepdims=True) + b_ref[0, 0]
    wgt_ref[...] = jax.nn.sigmoid(z).reshape(1, _BS, 1)


def _sc_body(x_hbm, seg_hbm, g_hbm, st_hbm, out_hbm,
             xbuf, bbuf, gbuf, stv, acc):
    cid = lax.axis_index("c")
    sid = lax.axis_index("s")
    wid = sid * _NC + cid

    pltpu.sync_copy(st_hbm, stv)

    zero = jnp.zeros((_L,), jnp.float32)

    def _zrow(i, _):
        for j in range(_D // _L):
            acc[i, pl.ds(j * _L, _L)] = zero
        return 0

    lax.fori_loop(0, _SEG_PER_TILE, _zrow, 0)

    base = wid * _SEG_PER_TILE
    a = stv[pl.ds(base, _L)][0]
    e = stv[pl.ds(base + _SEG_PER_TILE, _L)][0]
    a8 = (a >> 3) << 3
    nch = (e - a8 + _R - 1) // _R

    def _chunk(c, _):
        cs = pl.multiple_of(jnp.minimum(a8 + c * _R, _N - _R), 8)
        pltpu.sync_copy(x_hbm.at[pl.ds(cs, _R)], xbuf)
        pltpu.sync_copy(seg_hbm.at[pl.ds(cs, _R)], bbuf.at[pl.ds(0, _R)])
        pltpu.sync_copy(g_hbm.at[pl.ds(cs, _R)], gbuf.at[pl.ds(0, _R)])
        lo = jnp.maximum(a8 + c * _R, a)

        def _row(r):
            g = cs + r
            valid = jnp.logical_and(g >= lo, g < e)
            s_off = jnp.clip(bbuf[pl.ds(r, _L)][0] - base, 0,
                             _SEG_PER_TILE - 1)
            wk = jnp.where(valid, gbuf[pl.ds(r, _L)][0], 0.0)
            for j in range(_D // _L):
                plsc.addupdate(acc.at[s_off, pl.ds(j * _L, _L)],
                               xbuf[r, pl.ds(j * _L, _L)] * wk)

        plsc.parallel_loop(0, _R, unroll=8)(_row)
        return 0

    lax.fori_loop(0, nch, _chunk, 0)
    pltpu.sync_copy(acc, out_hbm.at[pl.ds(base, _SEG_PER_TILE)])


def kernel(x, batch, W, b):
    seg = batch.astype(jnp.int32)
    wr = W.reshape(1, _D).astype(jnp.float32)
    br = b.reshape(1, 1).astype(jnp.float32)

    starts, wgt = pl.pallas_call(
        _tc_fn,
        grid=(_G,),
        in_specs=[
            pl.BlockSpec((_BS, _D), lambda i: (i, 0)),
            pl.BlockSpec((1, _BS, 1), lambda i: (i, 0, 0)),
            pl.BlockSpec((1, _D), lambda i: (0, 0)),
            pl.BlockSpec((1, 1), lambda i: (0, 0)),
        ],
        out_specs=[
            pl.BlockSpec((1, _SP), lambda i: (0, 0)),
            pl.BlockSpec((1, _BS, 1), lambda i: (i, 0, 0)),
        ],
        out_shape=[
            jax.ShapeDtypeStruct((1, _SP), jnp.int32),
            jax.ShapeDtypeStruct((_G, _BS, 1), jnp.float32),
        ],
        compiler_params=pltpu.CompilerParams(
            dimension_semantics=("arbitrary",),
        ),
    )(x, seg.reshape(_G, _BS, 1), wr, br)

    mesh = plsc.VectorSubcoreMesh(
        core_axis_name="c", subcore_axis_name="s",
        num_cores=_NC, num_subcores=_NS)
    sc_fn = pl.kernel(
        _sc_body,
        out_type=jax.ShapeDtypeStruct((_S, _D), jnp.float32),
        mesh=mesh,
        scratch_types=[
            pltpu.VMEM((_R, _D), jnp.float32),
            pltpu.VMEM((_R + _L,), jnp.int32),
            pltpu.VMEM((_R + _L,), jnp.float32),
            pltpu.VMEM((_SP,), jnp.int32),
            pltpu.VMEM((_SEG_PER_TILE, _D), jnp.float32),
        ],
    )
    return sc_fn(x, seg, wgt.reshape(_N), starts.reshape(_SP))


# speedup vs baseline: 2.4362x; 1.0241x over previous
import jax
import jax.numpy as jnp
from jax import lax
from jax.experimental import pallas as pl
from jax.experimental.pallas import tpu as pltpu
from jax.experimental.pallas import tpu_sc as plsc

_N = 50000
_D = 256
_S = 512
_NC = 2
_NS = 16
_NW = _NC * _NS
_R = 112
_L = 16
_SEG_PER_TILE = _S // _NW
_SP = 128
_BS = 2000
_G = _N // _BS


def _tc_fn(xb_ref, seg_ref, w_ref, b_ref, st_ref, wgt_ref):
    i = pl.program_id(0)

    @pl.when(i == 0)
    def _init():
        st_ref[...] = jnp.zeros_like(st_ref)

    tile_of = seg_ref[0] >> 4
    lanes = lax.broadcasted_iota(jnp.int32, (_BS, _SP), 1)
    st_ref[...] += jnp.sum((tile_of < lanes).astype(jnp.int32), axis=0,
                           keepdims=True)
    xb = xb_ref[...]
    z = jnp.sum(xb * w_ref[...], axis=1, keepdims=True) + b_ref[0, 0]
    wgt_ref[...] = jax.nn.sigmoid(z).reshape(1, _BS, 1)


def _sc_body(x_hbm, seg_hbm, g_hbm, st_hbm, out_hbm,
             xbuf, bbuf, gbuf, stv, acc):
    cid = lax.axis_index("c")
    sid = lax.axis_index("s")
    wid = sid * _NC + cid

    pltpu.sync_copy(st_hbm, stv)

    zero = jnp.zeros((_L,), jnp.float32)

    def _zrow(i, _):
        for j in range(_D // _L):
            acc[i, pl.ds(j * _L, _L)] = zero
        return 0

    lax.fori_loop(0, _SEG_PER_TILE, _zrow, 0)

    base = wid * _SEG_PER_TILE
    a = stv[pl.ds(wid, _L)][0]
    e = stv[pl.ds(wid + 1, _L)][0]
    a8 = (a >> 3) << 3
    nch = (e - a8 + _R - 1) // _R

    def _chunk(c, _):
        cs = pl.multiple_of(jnp.minimum(a8 + c * _R, _N - _R), 8)
        pltpu.sync_copy(x_hbm.at[pl.ds(cs, _R)], xbuf)
        pltpu.sync_copy(seg_hbm.at[pl.ds(cs, _R)], bbuf.at[pl.ds(0, _R)])
        pltpu.sync_copy(g_hbm.at[pl.ds(cs, _R)], gbuf.at[pl.ds(0, _R)])
        lo = jnp.maximum(a8 + c * _R, a)

        def _row(r):
            g = cs + r
            valid = jnp.logical_and(g >= lo, g < e)
            s_off = jnp.clip(bbuf[pl.ds(r, _L)][0] - base, 0,
                             _SEG_PER_TILE - 1)
            wk = jnp.where(valid, gbuf[pl.ds(r, _L)][0], 0.0)
            for j in range(_D // _L):
                plsc.addupdate(acc.at[s_off, pl.ds(j * _L, _L)],
                               xbuf[r, pl.ds(j * _L, _L)] * wk)

        plsc.parallel_loop(0, _R, unroll=8)(_row)
        return 0

    lax.fori_loop(0, nch, _chunk, 0)
    pltpu.sync_copy(acc, out_hbm.at[pl.ds(base, _SEG_PER_TILE)])


def kernel(x, batch, W, b):
    seg = batch.astype(jnp.int32)
    wr = W.reshape(1, _D).astype(jnp.float32)
    br = b.reshape(1, 1).astype(jnp.float32)

    starts, wgt = pl.pallas_call(
        _tc_fn,
        grid=(_G,),
        in_specs=[
            pl.BlockSpec((_BS, _D), lambda i: (i, 0)),
            pl.BlockSpec((1, _BS, 1), lambda i: (i, 0, 0)),
            pl.BlockSpec((1, _D), lambda i: (0, 0)),
            pl.BlockSpec((1, 1), lambda i: (0, 0)),
        ],
        out_specs=[
            pl.BlockSpec((1, _SP), lambda i: (0, 0)),
            pl.BlockSpec((1, _BS, 1), lambda i: (i, 0, 0)),
        ],
        out_shape=[
            jax.ShapeDtypeStruct((1, _SP), jnp.int32),
            jax.ShapeDtypeStruct((_G, _BS, 1), jnp.float32),
        ],
        compiler_params=pltpu.CompilerParams(
            dimension_semantics=("arbitrary",),
        ),
    )(x, seg.reshape(_G, _BS, 1), wr, br)

    mesh = plsc.VectorSubcoreMesh(
        core_axis_name="c", subcore_axis_name="s",
        num_cores=_NC, num_subcores=_NS)
    sc_fn = pl.kernel(
        _sc_body,
        out_type=jax.ShapeDtypeStruct((_S, _D), jnp.float32),
        mesh=mesh,
        scratch_types=[
            pltpu.VMEM((_R, _D), jnp.float32),
            pltpu.VMEM((_R + _L,), jnp.int32),
            pltpu.VMEM((_R + _L,), jnp.float32),
            pltpu.VMEM((_SP,), jnp.int32),
            pltpu.VMEM((_SEG_PER_TILE, _D), jnp.float32),
        ],
    )
    return sc_fn(x, seg, wgt.reshape(_N), starts.reshape(_SP))


# speedup vs baseline: 2.4912x; 1.0226x over previous
import jax
import jax.numpy as jnp
from jax import lax
from jax.experimental import pallas as pl
from jax.experimental.pallas import tpu as pltpu
from jax.experimental.pallas import tpu_sc as plsc

_N = 50000
_D = 256
_S = 512
_NC = 2
_NS = 16
_NW = _NC * _NS
_R = 112
_L = 16
_SEG_PER_TILE = _S // _NW
_SP = 528
_BS = 2000
_G = _N // _BS


def _tc_fn(xb_ref, seg_ref, w_ref, b_ref, st_ref, wgt_ref):
    i = pl.program_id(0)

    @pl.when(i == 0)
    def _init():
        st_ref[...] = jnp.zeros_like(st_ref)

    seg = seg_ref[0]
    lanes = lax.broadcasted_iota(jnp.int32, (_BS, _SP), 1)
    st_ref[...] += jnp.sum((seg < lanes).astype(jnp.int32), axis=0,
                           keepdims=True)
    xb = xb_ref[...]
    z = jnp.sum(xb * w_ref[...], axis=1, keepdims=True) + b_ref[0, 0]
    wgt_ref[...] = jax.nn.sigmoid(z).reshape(1, _BS, 1)


def _sc_body(x_hbm, seg_hbm, g_hbm, st_hbm, out_hbm,
             xbuf, gbuf, stv, acc):
    cid = lax.axis_index("c")
    sid = lax.axis_index("s")
    wid = sid * _NC + cid

    pltpu.sync_copy(st_hbm, stv)

    base = wid * _SEG_PER_TILE
    nj = _D // _L

    def _seg(s_local, _):
        s_idx = base + s_local
        a = stv[pl.ds(s_idx, _L)][0]
        e = stv[pl.ds(s_idx + 1, _L)][0]
        a8 = (a >> 3) << 3
        nch = (e - a8 + _R - 1) // _R

        def _chunk(c, regs):
            cs = pl.multiple_of(jnp.minimum(a8 + c * _R, _N - _R), 8)
            pltpu.sync_copy(x_hbm.at[pl.ds(cs, _R)], xbuf)
            pltpu.sync_copy(g_hbm.at[pl.ds(cs, _R)], gbuf.at[pl.ds(0, _R)])
            lo = jnp.maximum(a8 + c * _R, a)

            def _row(r, regs):
                g = cs + r
                valid = jnp.logical_and(g >= lo, g < e)
                wk = jnp.where(valid, gbuf[pl.ds(r, _L)][0], 0.0)
                return tuple(regs[j] + xbuf[r, pl.ds(j * _L, _L)] * wk
                             for j in range(nj))

            return lax.fori_loop(0, _R, _row, regs, unroll=8)

        zero_regs = tuple(jnp.zeros((_L,), jnp.float32) for _ in range(nj))
        regs = lax.fori_loop(0, nch, _chunk, zero_regs)
        for j in range(nj):
            acc[s_local, pl.ds(j * _L, _L)] = regs[j]
        return 0

    lax.fori_loop(0, _SEG_PER_TILE, _seg, 0)
    pltpu.sync_copy(acc, out_hbm.at[pl.ds(base, _SEG_PER_TILE)])


def kernel(x, batch, W, b):
    seg = batch.astype(jnp.int32)
    wr = W.reshape(1, _D).astype(jnp.float32)
    br = b.reshape(1, 1).astype(jnp.float32)

    starts, wgt = pl.pallas_call(
        _tc_fn,
        grid=(_G,),
        in_specs=[
            pl.BlockSpec((_BS, _D), lambda i: (i, 0)),
            pl.BlockSpec((1, _BS, 1), lambda i: (i, 0, 0)),
            pl.BlockSpec((1, _D), lambda i: (0, 0)),
            pl.BlockSpec((1, 1), lambda i: (0, 0)),
        ],
        out_specs=[
            pl.BlockSpec((1, _SP), lambda i: (0, 0)),
            pl.BlockSpec((1, _BS, 1), lambda i: (i, 0, 0)),
        ],
        out_shape=[
            jax.ShapeDtypeStruct((1, _SP), jnp.int32),
            jax.ShapeDtypeStruct((_G, _BS, 1), jnp.float32),
        ],
        compiler_params=pltpu.CompilerParams(
            dimension_semantics=("arbitrary",),
        ),
    )(x, seg.reshape(_G, _BS, 1), wr, br)

    mesh = plsc.VectorSubcoreMesh(
        core_axis_name="c", subcore_axis_name="s",
        num_cores=_NC, num_subcores=_NS)
    sc_fn = pl.kernel(
        _sc_body,
        out_type=jax.ShapeDtypeStruct((_S, _D), jnp.float32),
        mesh=mesh,
        scratch_types=[
            pltpu.VMEM((_R, _D), jnp.float32),
            pltpu.VMEM((_R + _L,), jnp.float32),
            pltpu.VMEM((_SP,), jnp.int32),
            pltpu.VMEM((_SEG_PER_TILE, _D), jnp.float32),
        ],
    )
    return sc_fn(x, seg, wgt.reshape(_N), starts.reshape(_SP))
